# baseline (speedup 1.0000x reference)
import functools

import jax
import jax.numpy as jnp
from jax import lax
from jax.experimental import pallas as pl
from jax.experimental.pallas import tpu as pltpu
from jax.experimental.pallas import tpu_sc as plsc

NC = 2
NS = 16
NW = NC * NS
LANES = 16
CHUNK = 16


def _sc_kernel(B, S, D):
    mesh = plsc.VectorSubcoreMesh(
        core_axis_name="c", subcore_axis_name="s", num_cores=NC, num_subcores=NS
    )
    SW = S // NW
    n_chunks = SW // CHUNK
    KV = D // LANES

    @functools.partial(
        pl.kernel,
        out_type=jax.ShapeDtypeStruct((B, S, D), jnp.float32),
        mesh=mesh,
        scratch_types=[
            pltpu.VMEM((CHUNK,), jnp.int32),
            pltpu.VMEM((CHUNK, D), jnp.float32),
            pltpu.VMEM((CHUNK, D), jnp.float32),
            pltpu.VMEM((CHUNK, D), jnp.float32),
            pltpu.SemaphoreType.DMA,
        ],
    )
    def k(x_hbm, par_hbm, pe_hbm, out_hbm, idx_v, pelin_v, acc_v, g_v, sem):
        wid = lax.axis_index("s") * NC + lax.axis_index("c")
        s_base = wid * SW

        def chunk_body(j, carry):
            s0 = s_base + j * CHUNK
            pltpu.sync_copy(pe_hbm.at[pl.ds(s0, CHUNK)], pelin_v)
            for b in range(B):
                pltpu.sync_copy(par_hbm.at[b, pl.ds(s0, CHUNK)], idx_v)
                pltpu.sync_copy(x_hbm.at[b, pl.ds(s0, CHUNK)], acc_v)
                pltpu.async_copy(pe_hbm.at[idx_v], g_v, sem).wait()

                def row_body(r, c):
                    for kk in range(KV):
                        sl = (r, pl.ds(kk * LANES, LANES))
                        plsc.addupdate(acc_v.at[sl], pelin_v[sl] + g_v[sl])
                    return c

                lax.fori_loop(0, CHUNK, row_body, 0)
                pltpu.sync_copy(acc_v, out_hbm.at[b, pl.ds(s0, CHUNK)])
            return carry

        lax.fori_loop(0, n_chunks, chunk_body, 0)

    return k


def kernel(x, parents, pe_table):
    B, S, D = x.shape
    return _sc_kernel(B, S, D)(x, parents.astype(jnp.int32), pe_table)


# speedup vs baseline: 1.8608x; 1.8608x over previous
import functools

import jax
import jax.numpy as jnp
from jax import lax
from jax.experimental import pallas as pl
from jax.experimental.pallas import tpu as pltpu
from jax.experimental.pallas import tpu_sc as plsc

NC = 2
NS = 16
NW = NC * NS
LANES = 16
CHUNK = 16


def _sc_kernel(B, S, D):
    mesh = plsc.VectorSubcoreMesh(
        core_axis_name="c", subcore_axis_name="s", num_cores=NC, num_subcores=NS
    )
    SW = S // NW
    n_chunks = SW // CHUNK
    KV = D // LANES
    UNROLL = 2 * B
    n_iters = (n_chunks * B) // UNROLL

    @functools.partial(
        pl.kernel,
        out_type=jax.ShapeDtypeStruct((B, S, D), jnp.float32),
        mesh=mesh,
        scratch_types=[
            pltpu.VMEM((CHUNK,), jnp.int32),
            pltpu.VMEM((CHUNK,), jnp.int32),
            pltpu.VMEM((CHUNK, D), jnp.float32),
            pltpu.VMEM((CHUNK, D), jnp.float32),
            pltpu.VMEM((CHUNK, D), jnp.float32),
            pltpu.VMEM((CHUNK, D), jnp.float32),
            pltpu.VMEM((CHUNK, D), jnp.float32),
            pltpu.VMEM((CHUNK, D), jnp.float32),
            pltpu.SemaphoreType.DMA,
            pltpu.SemaphoreType.DMA,
            pltpu.SemaphoreType.DMA,
            pltpu.SemaphoreType.DMA,
            pltpu.SemaphoreType.DMA,
            pltpu.SemaphoreType.DMA,
            pltpu.SemaphoreType.DMA,
            pltpu.SemaphoreType.DMA,
        ],
    )
    def k(x_hbm, par_hbm, pe_hbm, out_hbm,
          idx0, idx1, xa0, xa1, ga0, ga1, pla0, pla1,
          si0, si1, sn0, sn1, so0, so1, sp0, sp1):
        idxb, xb, gb, plb = [idx0, idx1], [xa0, xa1], [ga0, ga1], [pla0, pla1]
        s_idx, s_in, s_out, s_pl = [si0, si1], [sn0, sn1], [so0, so1], [sp0, sp1]

        wid = lax.axis_index("s") * NC + lax.axis_index("c")
        s_base = wid * SW

        def seq0(i, u):
            return s_base + (2 * i + u // B) * CHUNK

        def start_idx(i, u):
            sl = u % 2
            pltpu.async_copy(
                par_hbm.at[u % B, pl.ds(seq0(i, u), CHUNK)], idxb[sl], s_idx[sl])

        def wait_idx(u):
            sl = u % 2
            pltpu.make_async_copy(
                par_hbm.at[0, pl.ds(0, CHUNK)], idxb[sl], s_idx[sl]).wait()

        def start_in(i, u):
            sl = u % 2
            s0 = seq0(i, u)
            pltpu.async_copy(x_hbm.at[u % B, pl.ds(s0, CHUNK)], xb[sl], s_in[sl])
            pltpu.async_copy(pe_hbm.at[idxb[sl]], gb[sl], s_in[sl])

        def wait_in(u):
            sl = u % 2
            pltpu.make_async_copy(
                x_hbm.at[0, pl.ds(0, CHUNK)], xb[sl], s_in[sl]).wait()
            pltpu.make_async_copy(pe_hbm.at[idxb[sl]], gb[sl], s_in[sl]).wait()

        def start_pl(i, jj):
            sl = jj % 2
            pltpu.async_copy(
                pe_hbm.at[pl.ds(s_base + (2 * i + jj) * CHUNK, CHUNK)],
                plb[sl], s_pl[sl])

        def wait_pl(jj):
            sl = jj % 2
            pltpu.make_async_copy(
                pe_hbm.at[pl.ds(0, CHUNK)], plb[sl], s_pl[sl]).wait()

        def start_out(i, u):
            sl = u % 2
            pltpu.async_copy(
                xb[sl], out_hbm.at[u % B, pl.ds(seq0(i, u), CHUNK)], s_out[sl])

        def wait_out(u):
            sl = u % 2
            pltpu.make_async_copy(
                xb[sl], out_hbm.at[0, pl.ds(0, CHUNK)], s_out[sl]).wait()

        def compute(u):
            sl = u % 2
            acc, g, pe_l = xb[sl], gb[sl], plb[(u // B) % 2]

            def row(r, c):
                for kk in range(KV):
                    s = (r, pl.ds(kk * LANES, LANES))
                    plsc.addupdate(acc.at[s], pe_l[s] + g[s])
                return c

            lax.fori_loop(0, CHUNK, row, 0)

        def step(i, u, has_next, has_next2, pl_prefetch):
            wait_in(u)
            if has_next:
                wait_idx(u + 1)
                wait_out(u + 1)
                start_in(i, u + 1)
            if has_next2:
                start_idx(i, u + 2)
            if u % B == 0:
                wait_pl(u // B)
                if pl_prefetch:
                    start_pl(i, u // B + 1)
            compute(u)
            start_out(i, u)

        start_idx(0, 0)
        start_pl(0, 0)
        wait_idx(0)
        start_in(0, 0)
        start_idx(0, 1)
        pltpu.async_copy(xb[1], out_hbm.at[1 % B, pl.ds(s_base, CHUNK)], s_out[1])

        def main_body(i, c):
            for u in range(UNROLL):
                step(i, u, True, True, True)
            return c

        lax.fori_loop(0, n_iters - 1, main_body, 0)

        i_last = n_iters - 1
        for u in range(UNROLL):
            t = UNROLL * i_last + u
            step(i_last, u,
                 t + 1 < n_iters * UNROLL,
                 t + 2 < n_iters * UNROLL,
                 2 * i_last + u // B + 1 < n_chunks)

        wait_out(0)
        wait_out(1)

    return k


def kernel(x, parents, pe_table):
    B, S, D = x.shape
    return _sc_kernel(B, S, D)(x, parents.astype(jnp.int32), pe_table)
